# baseline (speedup 1.0000x reference)
import functools

import jax
import jax.numpy as jnp
from jax.experimental import pallas as pl

NUM_BEAM = 32
NUM_BIN = 513
NUM_CHANNEL = 16
BATCH = 2048

FT = 8
BT = 512


def _beam_kernel(x_ref, m_ref, o_ref):
    for t in range(FT):
        lhs = jnp.concatenate([x_ref[:, 0, t, :], x_ref[:, 1, t, :]],
                              axis=-1)
        res = jnp.dot(lhs, m_ref[t],
                      preferred_element_type=jnp.float32)
        o_ref[:, 0, t, :] = res[:, :NUM_BEAM]
        o_ref[:, 1, t, :] = res[:, NUM_BEAM:]


@functools.partial(jax.jit, static_argnames=())
def kernel(input, W):
    B, _, F, C = input.shape
    N = W.shape[0]
    wr = jnp.transpose(W[:, 0], (1, 2, 0))
    wi = jnp.transpose(W[:, 1], (1, 2, 0))
    top = jnp.concatenate([wr, wi], axis=2)
    bot = jnp.concatenate([-wi, wr], axis=2)
    m = jnp.concatenate([top, bot], axis=1)

    nf = pl.cdiv(F, FT)
    nb = B // BT
    grid = (nf, nb)
    out = pl.pallas_call(
        _beam_kernel,
        grid=grid,
        in_specs=[
            pl.BlockSpec((BT, 2, FT, C), lambda f, b: (b, 0, f, 0)),
            pl.BlockSpec((FT, 2 * C, 2 * N), lambda f, b: (f, 0, 0)),
        ],
        out_specs=pl.BlockSpec((BT, 2, FT, N), lambda f, b: (b, 0, f, 0)),
        out_shape=jax.ShapeDtypeStruct((B, 2, F, N), jnp.float32),
    )(input, m)
    return out


# speedup vs baseline: 1.3286x; 1.3286x over previous
import functools

import jax
import jax.numpy as jnp
from jax.experimental import pallas as pl

NUM_BEAM = 32
NUM_BIN = 513
NUM_CHANNEL = 16
BATCH = 2048

G = 8
NG = (NUM_BIN + G - 1) // G
RT = 1024


def _beam_kernel(x_ref, dwr_ref, dwi_ref, o_ref):
    g = pl.program_id(0)
    x = x_ref[...]
    lane = jax.lax.broadcasted_iota(jnp.int32, x.shape, 1)
    x = jnp.where(lane < NUM_BIN * NUM_CHANNEL - g * (G * NUM_CHANNEL),
                  x, 0.0)
    up = jnp.roll(x, -1, axis=0)
    dn = jnp.roll(x, 1, axis=0)
    row = jax.lax.broadcasted_iota(jnp.int32, x.shape, 0)
    xp = jnp.where(row % 2 == 0, -up, dn)
    o_ref[...] = (
        jnp.dot(x, dwr_ref[0], preferred_element_type=jnp.float32)
        + jnp.dot(xp, dwi_ref[0], preferred_element_type=jnp.float32)
    )


@functools.partial(jax.jit, static_argnames=())
def kernel(input, W):
    B, _, F, C = input.shape
    N = W.shape[0]
    FP = NG * G
    wr = jnp.transpose(W[:, 0], (1, 2, 0))
    wi = jnp.transpose(W[:, 1], (1, 2, 0))
    pad = ((0, FP - F), (0, 0), (0, 0))
    wr = jnp.pad(wr, pad).reshape(NG, G, C, N)
    wi = jnp.pad(wi, pad).reshape(NG, G, C, N)
    eye = jnp.eye(G, dtype=jnp.float32)
    dwr = jnp.einsum('gfcn,fe->gfcen', wr, eye).reshape(NG, G * C, G * N)
    dwi = jnp.einsum('gfcn,fe->gfcen', wi, eye).reshape(NG, G * C, G * N)

    X = input.reshape(2 * B, F * C)
    nr = (2 * B) // RT
    out2 = pl.pallas_call(
        _beam_kernel,
        grid=(NG, nr),
        in_specs=[
            pl.BlockSpec((RT, G * C), lambda g, r: (r, g)),
            pl.BlockSpec((1, G * C, G * N), lambda g, r: (g, 0, 0)),
            pl.BlockSpec((1, G * C, G * N), lambda g, r: (g, 0, 0)),
        ],
        out_specs=pl.BlockSpec((RT, G * N), lambda g, r: (r, g)),
        out_shape=jax.ShapeDtypeStruct((2 * B, F * N), jnp.float32),
    )(X, dwr, dwi)
    return out2.reshape(B, 2, F, N)


# speedup vs baseline: 1.4115x; 1.0624x over previous
import functools

import jax
import jax.numpy as jnp
from jax.experimental import pallas as pl
from jax.experimental.pallas import tpu as pltpu

NUM_BEAM = 32
NUM_BIN = 513
NUM_CHANNEL = 16
BATCH = 2048

G = 8
NG = (NUM_BIN + G - 1) // G
RT = 128
FC = NUM_BIN * NUM_CHANNEL
FN = NUM_BIN * NUM_BEAM


def _beam_kernel(x_ref, dwr_ref, dwi_ref, o_ref):
    for g in range(NG):
        lo = g * G * NUM_CHANNEL
        w = min(G * NUM_CHANNEL, FC - lo)
        xg = x_ref[:, lo:lo + w]
        up = jnp.roll(xg, -1, axis=0)
        dn = jnp.roll(xg, 1, axis=0)
        row = jax.lax.broadcasted_iota(jnp.int32, xg.shape, 0)
        xp = jnp.where(row % 2 == 0, -up, dn)
        acc = (
            jnp.dot(xg, dwr_ref[g, :w, :], preferred_element_type=jnp.float32)
            + jnp.dot(xp, dwi_ref[g, :w, :], preferred_element_type=jnp.float32)
        )
        olo = g * G * NUM_BEAM
        ow = min(G * NUM_BEAM, FN - olo)
        o_ref[:, olo:olo + ow] = acc[:, :ow]


@functools.partial(jax.jit, static_argnames=())
def kernel(input, W):
    B, _, F, C = input.shape
    N = W.shape[0]
    FP = NG * G
    wr = jnp.transpose(W[:, 0], (1, 2, 0))
    wi = jnp.transpose(W[:, 1], (1, 2, 0))
    pad = ((0, FP - F), (0, 0), (0, 0))
    wr = jnp.pad(wr, pad).reshape(NG, G, C, N)
    wi = jnp.pad(wi, pad).reshape(NG, G, C, N)
    eye = jnp.eye(G, dtype=jnp.float32)
    dwr = jnp.einsum('gfcn,fe->gfcen', wr, eye).reshape(NG, G * C, G * N)
    dwi = jnp.einsum('gfcn,fe->gfcen', wi, eye).reshape(NG, G * C, G * N)

    X = input.reshape(2 * B, FC)
    nr = (2 * B) // RT
    out2 = pl.pallas_call(
        _beam_kernel,
        grid=(nr,),
        in_specs=[
            pl.BlockSpec((RT, FC), lambda r: (r, 0)),
            pl.BlockSpec(memory_space=pltpu.VMEM),
            pl.BlockSpec(memory_space=pltpu.VMEM),
        ],
        out_specs=pl.BlockSpec((RT, FN), lambda r: (r, 0)),
        out_shape=jax.ShapeDtypeStruct((2 * B, FN), jnp.float32),
    )(X, dwr, dwi)
    return out2.reshape(B, 2, F, N)


# speedup vs baseline: 9.9527x; 7.0514x over previous
import functools

import jax
import jax.numpy as jnp
from jax.experimental import pallas as pl

NUM_BEAM = 32
NUM_BIN = 513
NUM_CHANNEL = 16
BATCH = 2048

G = 3
GPC = 3
NGRP = NUM_BIN // G
NCELL = NGRP // GPC


def _beam_kernel(x_ref, lw_ref, o_ref):
    for k in range(GPC):
        xg = x_ref[:, k * G:(k + 1) * G]
        rhs = xg.reshape(2 * G * NUM_CHANNEL, BATCH)
        res = jnp.dot(lw_ref[0, k], rhs,
                      preferred_element_type=jnp.float32)
        o_ref[:, k * G:(k + 1) * G] = res.reshape(2, G, NUM_BEAM, BATCH)


@functools.partial(jax.jit, static_argnames=())
def kernel(input, W):
    B, _, F, C = input.shape
    N = W.shape[0]
    wrT = jnp.transpose(W[:, 0], (1, 0, 2))
    wiT = jnp.transpose(W[:, 1], (1, 0, 2))
    top = jnp.stack([wrT, -wiT], axis=2)
    bot = jnp.stack([wiT, wrT], axis=2)
    A = jnp.stack([top, bot], axis=1)
    A5 = A.reshape(NGRP, G, 2, N, 2, C)
    eye = jnp.eye(G, dtype=jnp.float32)
    LW = jnp.einsum('gfrnjc,fe->grfnjec', A5, eye)
    LW = LW.reshape(NCELL, GPC, G * 2 * N, G * 2 * C)

    XT = jnp.transpose(input, (1, 2, 3, 0))
    out = pl.pallas_call(
        _beam_kernel,
        grid=(NCELL,),
        in_specs=[
            pl.BlockSpec((2, GPC * G, C, B), lambda i: (0, i, 0, 0)),
            pl.BlockSpec((1, GPC, G * 2 * N, G * 2 * C),
                         lambda i: (i, 0, 0, 0)),
        ],
        out_specs=pl.BlockSpec((2, GPC * G, N, B), lambda i: (0, i, 0, 0)),
        out_shape=jax.ShapeDtypeStruct((2, F, N, B), jnp.float32),
    )(XT, LW)
    return jnp.transpose(out, (3, 0, 1, 2))


# speedup vs baseline: 10.2367x; 1.0285x over previous
import functools

import jax
import jax.numpy as jnp
from jax.experimental import pallas as pl

NUM_BEAM = 32
NUM_BIN = 513
NUM_CHANNEL = 16
BATCH = 2048

G = 3
GPC = 9
NGRP = NUM_BIN // G
NCELL = NGRP // GPC


def _beam_kernel(x_ref, lw_ref, o_ref):
    for k in range(GPC):
        xg = x_ref[:, k * G:(k + 1) * G]
        rhs = xg.reshape(2 * G * NUM_CHANNEL, BATCH)
        res = jnp.dot(lw_ref[0, k], rhs,
                      preferred_element_type=jnp.float32)
        o_ref[:, k * G:(k + 1) * G] = res.reshape(2, G, NUM_BEAM, BATCH)


@functools.partial(jax.jit, static_argnames=())
def kernel(input, W):
    B, _, F, C = input.shape
    N = W.shape[0]
    wrT = jnp.transpose(W[:, 0], (1, 0, 2))
    wiT = jnp.transpose(W[:, 1], (1, 0, 2))
    top = jnp.stack([wrT, -wiT], axis=2)
    bot = jnp.stack([wiT, wrT], axis=2)
    A = jnp.stack([top, bot], axis=1)
    A5 = A.reshape(NGRP, G, 2, N, 2, C)
    eye = jnp.eye(G, dtype=jnp.float32)
    LW = jnp.einsum('gfrnjc,fe->grfnjec', A5, eye)
    LW = LW.reshape(NCELL, GPC, G * 2 * N, G * 2 * C)

    XT = jnp.transpose(input, (1, 2, 3, 0))
    out = pl.pallas_call(
        _beam_kernel,
        grid=(NCELL,),
        in_specs=[
            pl.BlockSpec((2, GPC * G, C, B), lambda i: (0, i, 0, 0)),
            pl.BlockSpec((1, GPC, G * 2 * N, G * 2 * C),
                         lambda i: (i, 0, 0, 0)),
        ],
        out_specs=pl.BlockSpec((2, GPC * G, N, B), lambda i: (0, i, 0, 0)),
        out_shape=jax.ShapeDtypeStruct((2, F, N, B), jnp.float32),
    )(XT, LW)
    return jnp.transpose(out, (3, 0, 1, 2))


# speedup vs baseline: 21.3718x; 2.0878x over previous
import functools

import jax
import jax.numpy as jnp
from jax.experimental import pallas as pl

NUM_BEAM = 32
NUM_BIN = 513
NUM_CHANNEL = 16
BATCH = 2048

FT = 27
NCELL = NUM_BIN // FT


def _beam_kernel(x_ref, a_ref, o_ref):
    for t in range(FT):
        rhs = x_ref[:, t].reshape(2 * NUM_CHANNEL, BATCH)
        res = jnp.dot(a_ref[t], rhs,
                      preferred_element_type=jnp.float32)
        o_ref[:, t] = res.reshape(2, NUM_BEAM, BATCH)


@functools.partial(jax.jit, static_argnames=())
def kernel(input, W):
    B, _, F, C = input.shape
    N = W.shape[0]
    wrT = jnp.transpose(W[:, 0], (1, 0, 2))
    wiT = jnp.transpose(W[:, 1], (1, 0, 2))
    top = jnp.stack([wrT, -wiT], axis=2)
    bot = jnp.stack([wiT, wrT], axis=2)
    A = jnp.stack([top, bot], axis=1).reshape(F, 2 * N, 2 * C)

    XT = jnp.transpose(input, (1, 2, 3, 0))
    out = pl.pallas_call(
        _beam_kernel,
        grid=(NCELL,),
        in_specs=[
            pl.BlockSpec((2, FT, C, B), lambda i: (0, i, 0, 0)),
            pl.BlockSpec((FT, 2 * N, 2 * C), lambda i: (i, 0, 0)),
        ],
        out_specs=pl.BlockSpec((2, FT, N, B), lambda i: (0, i, 0, 0)),
        out_shape=jax.ShapeDtypeStruct((2, F, N, B), jnp.float32),
    )(XT, A)
    return jnp.transpose(out, (3, 0, 1, 2))
